# baseline (speedup 1.0000x reference)
import functools

import jax
import jax.numpy as jnp
from jax import lax
from jax.experimental import pallas as pl
from jax.experimental.pallas import tpu as pltpu
from jax.experimental.pallas import tpu_sc as plsc

N = 10000
N_PAD = 10240
E = 320000
D = 128
ALPHA = 0.2
EPS = 1e-5

NW = 32
C = 112
NCH = 90
E_PAD = NW * NCH * C
RPT = N_PAD // 16

_sc_mesh = plsc.VectorSubcoreMesh(core_axis_name="c", subcore_axis_name="s")


@functools.partial(
    pl.kernel,
    out_type=jax.ShapeDtypeStruct((2, 16, RPT), jnp.float32),
    mesh=_sc_mesh,
    scratch_types=[
        pltpu.VMEM((NCH, C), jnp.int32),
        pltpu.VMEM((NCH, C), jnp.float32),
        pltpu.VMEM_SHARED((N_PAD,), jnp.float32),
    ],
)
def _deg_kernel(dst_hbm, attr_hbm, zdeg_hbm, degp_hbm, dstbuf, attrbuf, deg_sh):
    c = lax.axis_index("c")
    s = lax.axis_index("s")
    wid = s * 2 + c
    pltpu.sync_copy(zdeg_hbm, deg_sh.at[pl.ds(s * RPT, RPT)])
    plsc.subcore_barrier()
    pltpu.sync_copy(dst_hbm.at[wid], dstbuf)
    pltpu.sync_copy(attr_hbm.at[wid], attrbuf)

    def chunk(j, carry):
        pltpu.sync_copy(attrbuf.at[j], deg_sh.at[dstbuf.at[j]], add=True)
        return carry

    lax.fori_loop(0, NCH, chunk, 0)
    plsc.subcore_barrier()
    pltpu.sync_copy(deg_sh.at[pl.ds(s * RPT, RPT)], degp_hbm.at[c, s])


def _lin_body(x_ref, w_ref, degp_ref, g_ref, dis_ref):
    h = jnp.dot(x_ref[...], w_ref[...], preferred_element_type=jnp.float32)
    deg = degp_ref[0] + degp_ref[1] + 1.0
    dis = lax.rsqrt(deg)
    g = h * dis
    g_ref[...] = g
    dis_ref[...] = dis


def _lin_call(x_pad, W, degp):
    return pl.pallas_call(
        _lin_body,
        grid=(16,),
        in_specs=[
            pl.BlockSpec((640, D), lambda i: (i, 0)),
            pl.BlockSpec((D, D), lambda i: (0, 0)),
            pl.BlockSpec((2, 640, 1), lambda i: (0, i, 0)),
        ],
        out_specs=[
            pl.BlockSpec((640, D), lambda i: (i, 0)),
            pl.BlockSpec((640, 1), lambda i: (i, 0)),
        ],
        out_shape=[
            jax.ShapeDtypeStruct((N_PAD, D), jnp.float32),
            jax.ShapeDtypeStruct((N_PAD, 1), jnp.float32),
        ],
    )(x_pad, W, degp)


@functools.partial(
    pl.kernel,
    out_type=jax.ShapeDtypeStruct((2, 16, RPT, D), jnp.float32),
    mesh=_sc_mesh,
    scratch_types=[
        pltpu.VMEM((2, 2, C), jnp.int32),
        pltpu.VMEM((2, C), jnp.float32),
        pltpu.VMEM((C, D), jnp.float32),
        pltpu.VMEM((C, D), jnp.float32),
        pltpu.VMEM((C, D), jnp.float32),
        pltpu.VMEM_SHARED((N_PAD, D), jnp.float32),
        pltpu.SemaphoreType.DMA,
        pltpu.SemaphoreType.DMA,
        pltpu.SemaphoreType.DMA,
        pltpu.SemaphoreType.DMA,
    ],
)
def _scat_kernel(gp_hbm, sd_hbm, attr_hbm, zrows_hbm, sp_hbm,
                 idxbuf, attrbuf, rowsi0, rowsi1, rowsf, s_sh,
                 sem_i, sem_g0, sem_g1, sem_s):
    c = lax.axis_index("c")
    s = lax.axis_index("s")
    wid = s * 2 + c
    pltpu.sync_copy(zrows_hbm, s_sh.at[pl.ds(s * RPT, RPT)])
    plsc.subcore_barrier()

    rowsi = (rowsi0, rowsi1)
    sem_g = (sem_g0, sem_g1)

    def scale(p, bufi):
        def scale_grp(grp, carry2):
            base = grp * 16
            avs = attrbuf[p, pl.ds(base, 16)]
            for dr in range(16):
                av = jnp.full((16,), avs[dr], jnp.float32)
                for f in range(8):
                    sl = pl.ds(f * 16, 16)
                    rowsf[base + dr, sl] = bufi[base + dr, sl] * av
            return carry2

        lax.fori_loop(0, C // 16, scale_grp, 0)

    pltpu.async_copy(sd_hbm.at[wid, 0], idxbuf.at[0], sem_i).wait()
    pltpu.async_copy(attr_hbm.at[wid, 0], attrbuf.at[0], sem_i).wait()
    pltpu.async_copy(gp_hbm.at[idxbuf.at[0, 0]], rowsi[0], sem_g[0])

    def two(q, carry):
        for u in (0, 1):
            b = u
            nb = 1 - u
            j = q * 2 + u
            pltpu.make_async_copy(
                gp_hbm.at[idxbuf.at[b, 0]], rowsi[b], sem_g[b]).wait()

            if u == 0:
                @pl.when(q > 0)
                def _():
                    pltpu.make_async_copy(
                        rowsf, s_sh.at[idxbuf.at[1, 1]], sem_s).wait()
            else:
                pltpu.make_async_copy(
                    rowsf, s_sh.at[idxbuf.at[0, 1]], sem_s).wait()

            @pl.when(j < NCH - 1)
            def _():
                pltpu.async_copy(
                    sd_hbm.at[wid, j + 1], idxbuf.at[nb], sem_i).wait()
                pltpu.async_copy(
                    attr_hbm.at[wid, j + 1], attrbuf.at[nb], sem_i).wait()
                pltpu.async_copy(gp_hbm.at[idxbuf.at[nb, 0]], rowsi[nb],
                                 sem_g[nb])

            scale(b, rowsi[b])
            pltpu.async_copy(rowsf, s_sh.at[idxbuf.at[b, 1]], sem_s,
                             add=True)

        return carry

    lax.fori_loop(0, NCH // 2, two, 0)
    pltpu.make_async_copy(
        rowsf, s_sh.at[idxbuf.at[(NCH - 1) % 2, 1]], sem_s).wait()
    plsc.subcore_barrier()
    pltpu.sync_copy(s_sh.at[pl.ds(s * RPT, RPT)], sp_hbm.at[c, s])


def _agg_body(g_ref, sp_ref, dis_ref, b_ref, agg_ref, sum_ref, sumsq_ref):
    i = pl.program_id(0)
    sblk = sp_ref[0] + sp_ref[1]
    agg = dis_ref[...] * (g_ref[...] + sblk) + b_ref[...]
    agg_ref[...] = agg
    s0 = jnp.sum(agg, axis=0, keepdims=True)
    s1 = jnp.sum(agg * agg, axis=0, keepdims=True)

    @pl.when(i == 0)
    def _():
        sum_ref[...] = s0
        sumsq_ref[...] = s1

    @pl.when(i > 0)
    def _():
        sum_ref[...] += s0
        sumsq_ref[...] += s1


def _agg_call(g, sp, dis, b2):
    return pl.pallas_call(
        _agg_body,
        grid=(10,),
        in_specs=[
            pl.BlockSpec((1000, D), lambda i: (i, 0)),
            pl.BlockSpec((2, 1000, D), lambda i: (0, i, 0)),
            pl.BlockSpec((1000, 1), lambda i: (i, 0)),
            pl.BlockSpec((1, D), lambda i: (0, 0)),
        ],
        out_specs=[
            pl.BlockSpec((1000, D), lambda i: (i, 0)),
            pl.BlockSpec((1, D), lambda i: (0, 0)),
            pl.BlockSpec((1, D), lambda i: (0, 0)),
        ],
        out_shape=[
            jax.ShapeDtypeStruct((N, D), jnp.float32),
            jax.ShapeDtypeStruct((1, D), jnp.float32),
            jax.ShapeDtypeStruct((1, D), jnp.float32),
        ],
    )(g, sp, dis, b2)


def _bn_body(agg_ref, sum_ref, sumsq_ref, gamma_ref, beta_ref, o_ref):
    mean = sum_ref[...] * (1.0 / N)
    var = sumsq_ref[...] * (1.0 / N) - mean * mean
    inv = lax.rsqrt(var + EPS)
    hn = (agg_ref[...] - mean) * inv * gamma_ref[...] + beta_ref[...]
    o_ref[...] = jnp.where(hn >= 0, hn, ALPHA * hn)


def _bn_call(agg, s0, s1, gamma2, beta2):
    return pl.pallas_call(
        _bn_body,
        grid=(10,),
        in_specs=[
            pl.BlockSpec((1000, D), lambda i: (i, 0)),
            pl.BlockSpec((1, D), lambda i: (0, 0)),
            pl.BlockSpec((1, D), lambda i: (0, 0)),
            pl.BlockSpec((1, D), lambda i: (0, 0)),
            pl.BlockSpec((1, D), lambda i: (0, 0)),
        ],
        out_specs=pl.BlockSpec((1000, D), lambda i: (i, 0)),
        out_shape=jax.ShapeDtypeStruct((N, D), jnp.float32),
    )(agg, s0, s1, gamma2, beta2)


def kernel(x, edge_idx, edge_attr, W, b, gamma, beta):
    src = edge_idx[0]
    dst = edge_idx[1]
    pad = E_PAD - E
    ar = jnp.arange(pad, dtype=jnp.int32)
    src_p = jnp.concatenate([src, ar % N]).reshape(NW, NCH, C)
    dst_p = jnp.concatenate([dst, N + ar % (N_PAD - N)]).reshape(NW, NCH, C)
    attr_p = jnp.concatenate(
        [edge_attr, jnp.zeros((pad,), jnp.float32)]).reshape(NW, NCH, C)
    zdeg = jnp.zeros((RPT,), jnp.float32)
    zrows = jnp.zeros((RPT, D), jnp.float32)
    x_pad = jnp.pad(x, ((0, N_PAD - N), (0, 0)))

    degp = _deg_kernel(dst_p, attr_p, zdeg).reshape(2, N_PAD, 1)
    g, dis = _lin_call(x_pad, W, degp)
    sd_p = jnp.stack([src_p, dst_p], axis=2)
    sp = _scat_kernel(g, sd_p, attr_p, zrows).reshape(2, N_PAD, D)
    agg, s0, s1 = _agg_call(g, sp, dis, b.reshape(1, D))
    return _bn_call(agg, s0, s1, gamma.reshape(1, D), beta.reshape(1, D))


# speedup vs baseline: 1.0277x; 1.0277x over previous
import functools

import jax
import jax.numpy as jnp
from jax import lax
from jax.experimental import pallas as pl
from jax.experimental.pallas import tpu as pltpu
from jax.experimental.pallas import tpu_sc as plsc

N = 10000
N_PAD = 10240
E = 320000
D = 128
ALPHA = 0.2
EPS = 1e-5

NW = 32
C = 112
NCH = 90
E_PAD = NW * NCH * C
RPT = N_PAD // 16

_sc_mesh = plsc.VectorSubcoreMesh(core_axis_name="c", subcore_axis_name="s")


@functools.partial(
    pl.kernel,
    out_type=jax.ShapeDtypeStruct((2, 16, RPT), jnp.float32),
    mesh=_sc_mesh,
    scratch_types=[
        pltpu.VMEM((NCH, C), jnp.int32),
        pltpu.VMEM((NCH, C), jnp.float32),
        pltpu.VMEM_SHARED((N_PAD,), jnp.float32),
    ],
)
def _deg_kernel(dst_hbm, attr_hbm, zdeg_hbm, degp_hbm, dstbuf, attrbuf, deg_sh):
    c = lax.axis_index("c")
    s = lax.axis_index("s")
    wid = s * 2 + c
    pltpu.sync_copy(zdeg_hbm, deg_sh.at[pl.ds(s * RPT, RPT)])
    plsc.subcore_barrier()
    pltpu.sync_copy(dst_hbm.at[wid], dstbuf)
    pltpu.sync_copy(attr_hbm.at[wid], attrbuf)

    def chunk(j, carry):
        pltpu.sync_copy(attrbuf.at[j], deg_sh.at[dstbuf.at[j]], add=True)
        return carry

    lax.fori_loop(0, NCH, chunk, 0)
    plsc.subcore_barrier()
    pltpu.sync_copy(deg_sh.at[pl.ds(s * RPT, RPT)], degp_hbm.at[c, s])


def _lin_body(x_ref, w_ref, degp_ref, g_ref, dis_ref):
    h = jnp.dot(x_ref[...], w_ref[...], preferred_element_type=jnp.float32)
    deg = degp_ref[0] + degp_ref[1] + 1.0
    dis = lax.rsqrt(deg)
    g = h * dis
    g_ref[...] = g
    dis_ref[...] = dis


def _lin_call(x_pad, W, degp):
    return pl.pallas_call(
        _lin_body,
        grid=(16,),
        in_specs=[
            pl.BlockSpec((640, D), lambda i: (i, 0)),
            pl.BlockSpec((D, D), lambda i: (0, 0)),
            pl.BlockSpec((2, 640, 1), lambda i: (0, i, 0)),
        ],
        out_specs=[
            pl.BlockSpec((640, D), lambda i: (i, 0)),
            pl.BlockSpec((640, 1), lambda i: (i, 0)),
        ],
        out_shape=[
            jax.ShapeDtypeStruct((N_PAD, D), jnp.float32),
            jax.ShapeDtypeStruct((N_PAD, 1), jnp.float32),
        ],
    )(x_pad, W, degp)


@functools.partial(
    pl.kernel,
    out_type=jax.ShapeDtypeStruct((2, 16, RPT, D), jnp.float32),
    mesh=_sc_mesh,
    scratch_types=[
        pltpu.VMEM((2, C), jnp.int32),
        pltpu.VMEM((2, C), jnp.int32),
        pltpu.VMEM((2, C), jnp.float32),
        pltpu.VMEM((C, D), jnp.float32),
        pltpu.VMEM((C, D), jnp.float32),
        pltpu.VMEM((C, D), jnp.float32),
        pltpu.VMEM_SHARED((N_PAD, D), jnp.float32),
        pltpu.SemaphoreType.DMA,
        pltpu.SemaphoreType.DMA,
        pltpu.SemaphoreType.DMA,
        pltpu.SemaphoreType.DMA,
    ],
)
def _scat_kernel(gp_hbm, src_hbm, dst_hbm, attr_hbm, zrows_hbm, sp_hbm,
                 srcbuf, dstbuf, attrbuf, rowsi0, rowsi1, rowsf, s_sh,
                 sem_i, sem_g0, sem_g1, sem_s):
    c = lax.axis_index("c")
    s = lax.axis_index("s")
    wid = s * 2 + c
    pltpu.sync_copy(zrows_hbm, s_sh.at[pl.ds(s * RPT, RPT)])
    plsc.subcore_barrier()

    rowsi = (rowsi0, rowsi1)
    sem_g = (sem_g0, sem_g1)

    def scale(p, bufi):
        def scale_grp(grp, carry2):
            base = grp * 16
            avs = attrbuf[p, pl.ds(base, 16)]
            for dr in range(16):
                av = jnp.full((16,), avs[dr], jnp.float32)
                for f in range(8):
                    sl = pl.ds(f * 16, 16)
                    rowsf[base + dr, sl] = bufi[base + dr, sl] * av
            return carry2

        lax.fori_loop(0, C // 16, scale_grp, 0)

    pltpu.async_copy(src_hbm.at[wid, 0], srcbuf.at[0], sem_i).wait()
    pltpu.async_copy(src_hbm.at[wid, 1], srcbuf.at[1], sem_i).wait()
    pltpu.async_copy(attr_hbm.at[wid, 0], attrbuf.at[0], sem_i).wait()
    pltpu.async_copy(attr_hbm.at[wid, 1], attrbuf.at[1], sem_i).wait()
    pltpu.async_copy(dst_hbm.at[wid, 0], dstbuf.at[0], sem_i).wait()
    pltpu.async_copy(gp_hbm.at[srcbuf.at[0]], rowsi[0], sem_g[0])
    pltpu.async_copy(gp_hbm.at[srcbuf.at[1]], rowsi[1], sem_g[1])

    def two(q, carry):
        for u in (0, 1):
            b = u
            nb = 1 - u
            j = q * 2 + u
            pltpu.make_async_copy(
                gp_hbm.at[srcbuf.at[b]], rowsi[b], sem_g[b]).wait()

            if u == 0:
                @pl.when(q > 0)
                def _():
                    pltpu.make_async_copy(
                        rowsf, s_sh.at[dstbuf.at[1]], sem_s).wait()
            else:
                pltpu.make_async_copy(
                    rowsf, s_sh.at[dstbuf.at[0]], sem_s).wait()

            @pl.when(j < NCH - 1)
            def _():
                pltpu.async_copy(
                    dst_hbm.at[wid, j + 1], dstbuf.at[nb], sem_i).wait()

            scale(b, rowsi[b])
            pltpu.async_copy(rowsf, s_sh.at[dstbuf.at[b]], sem_s,
                             add=True)

            @pl.when(j < NCH - 2)
            def _():
                pltpu.async_copy(
                    src_hbm.at[wid, j + 2], srcbuf.at[b], sem_i).wait()
                pltpu.async_copy(
                    attr_hbm.at[wid, j + 2], attrbuf.at[b], sem_i).wait()
                pltpu.async_copy(gp_hbm.at[srcbuf.at[b]], rowsi[b],
                                 sem_g[b])

        return carry

    lax.fori_loop(0, NCH // 2, two, 0)
    pltpu.make_async_copy(
        rowsf, s_sh.at[dstbuf.at[(NCH - 1) % 2]], sem_s).wait()
    plsc.subcore_barrier()
    pltpu.sync_copy(s_sh.at[pl.ds(s * RPT, RPT)], sp_hbm.at[c, s])


def _agg_body(g_ref, sp_ref, dis_ref, b_ref, agg_ref, sum_ref, sumsq_ref):
    i = pl.program_id(0)
    sblk = sp_ref[0] + sp_ref[1]
    agg = dis_ref[...] * (g_ref[...] + sblk) + b_ref[...]
    agg_ref[...] = agg
    s0 = jnp.sum(agg, axis=0, keepdims=True)
    s1 = jnp.sum(agg * agg, axis=0, keepdims=True)

    @pl.when(i == 0)
    def _():
        sum_ref[...] = s0
        sumsq_ref[...] = s1

    @pl.when(i > 0)
    def _():
        sum_ref[...] += s0
        sumsq_ref[...] += s1


def _agg_call(g, sp, dis, b2):
    return pl.pallas_call(
        _agg_body,
        grid=(10,),
        in_specs=[
            pl.BlockSpec((1000, D), lambda i: (i, 0)),
            pl.BlockSpec((2, 1000, D), lambda i: (0, i, 0)),
            pl.BlockSpec((1000, 1), lambda i: (i, 0)),
            pl.BlockSpec((1, D), lambda i: (0, 0)),
        ],
        out_specs=[
            pl.BlockSpec((1000, D), lambda i: (i, 0)),
            pl.BlockSpec((1, D), lambda i: (0, 0)),
            pl.BlockSpec((1, D), lambda i: (0, 0)),
        ],
        out_shape=[
            jax.ShapeDtypeStruct((N, D), jnp.float32),
            jax.ShapeDtypeStruct((1, D), jnp.float32),
            jax.ShapeDtypeStruct((1, D), jnp.float32),
        ],
    )(g, sp, dis, b2)


def _bn_body(agg_ref, sum_ref, sumsq_ref, gamma_ref, beta_ref, o_ref):
    mean = sum_ref[...] * (1.0 / N)
    var = sumsq_ref[...] * (1.0 / N) - mean * mean
    inv = lax.rsqrt(var + EPS)
    hn = (agg_ref[...] - mean) * inv * gamma_ref[...] + beta_ref[...]
    o_ref[...] = jnp.where(hn >= 0, hn, ALPHA * hn)


def _bn_call(agg, s0, s1, gamma2, beta2):
    return pl.pallas_call(
        _bn_body,
        grid=(10,),
        in_specs=[
            pl.BlockSpec((1000, D), lambda i: (i, 0)),
            pl.BlockSpec((1, D), lambda i: (0, 0)),
            pl.BlockSpec((1, D), lambda i: (0, 0)),
            pl.BlockSpec((1, D), lambda i: (0, 0)),
            pl.BlockSpec((1, D), lambda i: (0, 0)),
        ],
        out_specs=pl.BlockSpec((1000, D), lambda i: (i, 0)),
        out_shape=jax.ShapeDtypeStruct((N, D), jnp.float32),
    )(agg, s0, s1, gamma2, beta2)


def kernel(x, edge_idx, edge_attr, W, b, gamma, beta):
    src = edge_idx[0]
    dst = edge_idx[1]
    pad = E_PAD - E
    ar = jnp.arange(pad, dtype=jnp.int32)
    src_p = jnp.concatenate([src, ar % N]).reshape(NW, NCH, C)
    dst_p = jnp.concatenate([dst, N + ar % (N_PAD - N)]).reshape(NW, NCH, C)
    attr_p = jnp.concatenate(
        [edge_attr, jnp.zeros((pad,), jnp.float32)]).reshape(NW, NCH, C)
    zdeg = jnp.zeros((RPT,), jnp.float32)
    zrows = jnp.zeros((RPT, D), jnp.float32)
    x_pad = jnp.pad(x, ((0, N_PAD - N), (0, 0)))

    degp = _deg_kernel(dst_p, attr_p, zdeg).reshape(2, N_PAD, 1)
    g, dis = _lin_call(x_pad, W, degp)
    sp = _scat_kernel(g, src_p, dst_p, attr_p, zrows).reshape(2, N_PAD, D)
    agg, s0, s1 = _agg_call(g, sp, dis, b.reshape(1, D))
    return _bn_call(agg, s0, s1, gamma.reshape(1, D), beta.reshape(1, D))


# speedup vs baseline: 1.3581x; 1.3215x over previous
import functools

import jax
import jax.numpy as jnp
from jax import lax
from jax.experimental import pallas as pl
from jax.experimental.pallas import tpu as pltpu
from jax.experimental.pallas import tpu_sc as plsc

N = 10000
N_PAD = 10240
E = 320000
D = 128
ALPHA = 0.2
EPS = 1e-5

NW = 32
C = 112
NCH = 90
E_PAD = NW * NCH * C
RPT = N_PAD // 16

_sc_mesh = plsc.VectorSubcoreMesh(core_axis_name="c", subcore_axis_name="s")


@functools.partial(
    pl.kernel,
    out_type=jax.ShapeDtypeStruct((2, 16, RPT), jnp.float32),
    mesh=_sc_mesh,
    scratch_types=[
        pltpu.VMEM((NCH, C), jnp.int32),
        pltpu.VMEM((NCH, C), jnp.float32),
        pltpu.VMEM_SHARED((N_PAD,), jnp.float32),
    ],
)
def _deg_kernel(dst_hbm, attr_hbm, zdeg_hbm, degp_hbm, dstbuf, attrbuf, deg_sh):
    c = lax.axis_index("c")
    s = lax.axis_index("s")
    wid = s * 2 + c
    pltpu.sync_copy(zdeg_hbm, deg_sh.at[pl.ds(s * RPT, RPT)])
    plsc.subcore_barrier()
    pltpu.sync_copy(dst_hbm.at[wid], dstbuf)
    pltpu.sync_copy(attr_hbm.at[wid], attrbuf)

    def chunk(j, carry):
        pltpu.sync_copy(attrbuf.at[j], deg_sh.at[dstbuf.at[j]], add=True)
        return carry

    lax.fori_loop(0, NCH, chunk, 0)
    plsc.subcore_barrier()
    pltpu.sync_copy(deg_sh.at[pl.ds(s * RPT, RPT)], degp_hbm.at[c, s])


def _lin_body(x_ref, w_ref, degp_ref, g_ref, dis_ref):
    h = jnp.dot(x_ref[...], w_ref[...], preferred_element_type=jnp.float32)
    deg = degp_ref[0] + degp_ref[1] + 1.0
    dis = lax.rsqrt(deg)
    g = h * dis
    g_ref[...] = g
    dis_ref[...] = dis


def _lin_call(x_pad, W, degp):
    return pl.pallas_call(
        _lin_body,
        grid=(16,),
        in_specs=[
            pl.BlockSpec((640, D), lambda i: (i, 0)),
            pl.BlockSpec((D, D), lambda i: (0, 0)),
            pl.BlockSpec((2, 640, 1), lambda i: (0, i, 0)),
        ],
        out_specs=[
            pl.BlockSpec((640, D), lambda i: (i, 0)),
            pl.BlockSpec((640, 1), lambda i: (i, 0)),
        ],
        out_shape=[
            jax.ShapeDtypeStruct((N_PAD, D), jnp.float32),
            jax.ShapeDtypeStruct((N_PAD, 1), jnp.float32),
        ],
    )(x_pad, W, degp)


@functools.partial(
    pl.kernel,
    out_type=jax.ShapeDtypeStruct((2, 16, RPT, D), jnp.float32),
    mesh=_sc_mesh,
    scratch_types=[
        pltpu.VMEM((2, C), jnp.int32),
        pltpu.VMEM((2, C), jnp.int32),
        pltpu.VMEM((2, C), jnp.float32),
        pltpu.VMEM((C, D), jnp.float32),
        pltpu.VMEM((C, D), jnp.float32),
        pltpu.VMEM((C, D), jnp.float32),
        pltpu.VMEM_SHARED((N_PAD, D), jnp.float32),
        pltpu.SemaphoreType.DMA,
        pltpu.SemaphoreType.DMA,
        pltpu.SemaphoreType.DMA,
        pltpu.SemaphoreType.DMA,
        pltpu.SemaphoreType.DMA,
        pltpu.SemaphoreType.DMA,
        pltpu.SemaphoreType.DMA,
        pltpu.SemaphoreType.DMA,
        pltpu.SemaphoreType.DMA,
    ],
)
def _scat_kernel(gp_hbm, src_hbm, dst_hbm, attr_hbm, zrows_hbm, sp_hbm,
                 srcbuf, dstbuf, attrbuf, rowsi0, rowsi1, rowsf, s_sh,
                 sem_src0, sem_src1, sem_dst0, sem_dst1, sem_attr0,
                 sem_attr1, sem_g0, sem_g1, sem_s):
    c = lax.axis_index("c")
    s = lax.axis_index("s")
    wid = s * 2 + c
    pltpu.sync_copy(zrows_hbm, s_sh.at[pl.ds(s * RPT, RPT)])
    plsc.subcore_barrier()

    rowsi = (rowsi0, rowsi1)
    sem_g = (sem_g0, sem_g1)
    sem_src = (sem_src0, sem_src1)
    sem_dst = (sem_dst0, sem_dst1)
    sem_attr = (sem_attr0, sem_attr1)

    def scale(p, bufi):
        def scale_grp(grp, carry2):
            base = grp * 16
            avs = attrbuf[p, pl.ds(base, 16)]
            for dr in range(16):
                av = jnp.full((16,), avs[dr], jnp.float32)
                for f in range(8):
                    sl = pl.ds(f * 16, 16)
                    rowsf[base + dr, sl] = bufi[base + dr, sl] * av
            return carry2

        lax.fori_loop(0, C // 16, scale_grp, 0)

    pltpu.async_copy(src_hbm.at[wid, 0], srcbuf.at[0], sem_src[0])
    pltpu.async_copy(src_hbm.at[wid, 1], srcbuf.at[1], sem_src[1])
    pltpu.async_copy(attr_hbm.at[wid, 0], attrbuf.at[0], sem_attr[0])
    pltpu.async_copy(attr_hbm.at[wid, 1], attrbuf.at[1], sem_attr[1])
    pltpu.async_copy(dst_hbm.at[wid, 0], dstbuf.at[0], sem_dst[0])
    pltpu.make_async_copy(
        src_hbm.at[wid, 0], srcbuf.at[0], sem_src[0]).wait()
    pltpu.async_copy(gp_hbm.at[srcbuf.at[0]], rowsi[0], sem_g[0])
    pltpu.make_async_copy(
        src_hbm.at[wid, 1], srcbuf.at[1], sem_src[1]).wait()
    pltpu.async_copy(gp_hbm.at[srcbuf.at[1]], rowsi[1], sem_g[1])

    def two(q, carry):
        for u in (0, 1):
            b = u
            nb = 1 - u
            j = q * 2 + u
            pltpu.make_async_copy(
                gp_hbm.at[srcbuf.at[b]], rowsi[b], sem_g[b]).wait()

            @pl.when(j < NCH - 2)
            def _():
                pltpu.async_copy(
                    src_hbm.at[wid, j + 2], srcbuf.at[b], sem_src[b])

            if u == 0:
                @pl.when(q > 0)
                def _():
                    pltpu.make_async_copy(
                        rowsf, s_sh.at[dstbuf.at[1]], sem_s).wait()
            else:
                pltpu.make_async_copy(
                    rowsf, s_sh.at[dstbuf.at[0]], sem_s).wait()

            @pl.when(j < NCH - 1)
            def _():
                pltpu.async_copy(
                    dst_hbm.at[wid, j + 1], dstbuf.at[nb], sem_dst[nb])

            pltpu.make_async_copy(
                attr_hbm.at[wid, j], attrbuf.at[b], sem_attr[b]).wait()
            scale(b, rowsi[b])
            pltpu.make_async_copy(
                dst_hbm.at[wid, j], dstbuf.at[b], sem_dst[b]).wait()
            pltpu.async_copy(rowsf, s_sh.at[dstbuf.at[b]], sem_s,
                             add=True)

            @pl.when(j < NCH - 2)
            def _():
                pltpu.make_async_copy(
                    src_hbm.at[wid, j + 2], srcbuf.at[b], sem_src[b]).wait()
                pltpu.async_copy(gp_hbm.at[srcbuf.at[b]], rowsi[b],
                                 sem_g[b])
                pltpu.async_copy(
                    attr_hbm.at[wid, j + 2], attrbuf.at[b], sem_attr[b])

        return carry

    lax.fori_loop(0, NCH // 2, two, 0)
    pltpu.make_async_copy(
        rowsf, s_sh.at[dstbuf.at[(NCH - 1) % 2]], sem_s).wait()
    plsc.subcore_barrier()
    pltpu.sync_copy(s_sh.at[pl.ds(s * RPT, RPT)], sp_hbm.at[c, s])


def _agg_body(g_ref, sp_ref, dis_ref, b_ref, agg_ref, sum_ref, sumsq_ref):
    i = pl.program_id(0)
    sblk = sp_ref[0] + sp_ref[1]
    agg = dis_ref[...] * (g_ref[...] + sblk) + b_ref[...]
    agg_ref[...] = agg
    s0 = jnp.sum(agg, axis=0, keepdims=True)
    s1 = jnp.sum(agg * agg, axis=0, keepdims=True)

    @pl.when(i == 0)
    def _():
        sum_ref[...] = s0
        sumsq_ref[...] = s1

    @pl.when(i > 0)
    def _():
        sum_ref[...] += s0
        sumsq_ref[...] += s1


def _agg_call(g, sp, dis, b2):
    return pl.pallas_call(
        _agg_body,
        grid=(10,),
        in_specs=[
            pl.BlockSpec((1000, D), lambda i: (i, 0)),
            pl.BlockSpec((2, 1000, D), lambda i: (0, i, 0)),
            pl.BlockSpec((1000, 1), lambda i: (i, 0)),
            pl.BlockSpec((1, D), lambda i: (0, 0)),
        ],
        out_specs=[
            pl.BlockSpec((1000, D), lambda i: (i, 0)),
            pl.BlockSpec((1, D), lambda i: (0, 0)),
            pl.BlockSpec((1, D), lambda i: (0, 0)),
        ],
        out_shape=[
            jax.ShapeDtypeStruct((N, D), jnp.float32),
            jax.ShapeDtypeStruct((1, D), jnp.float32),
            jax.ShapeDtypeStruct((1, D), jnp.float32),
        ],
    )(g, sp, dis, b2)


def _bn_body(agg_ref, sum_ref, sumsq_ref, gamma_ref, beta_ref, o_ref):
    mean = sum_ref[...] * (1.0 / N)
    var = sumsq_ref[...] * (1.0 / N) - mean * mean
    inv = lax.rsqrt(var + EPS)
    hn = (agg_ref[...] - mean) * inv * gamma_ref[...] + beta_ref[...]
    o_ref[...] = jnp.where(hn >= 0, hn, ALPHA * hn)


def _bn_call(agg, s0, s1, gamma2, beta2):
    return pl.pallas_call(
        _bn_body,
        grid=(10,),
        in_specs=[
            pl.BlockSpec((1000, D), lambda i: (i, 0)),
            pl.BlockSpec((1, D), lambda i: (0, 0)),
            pl.BlockSpec((1, D), lambda i: (0, 0)),
            pl.BlockSpec((1, D), lambda i: (0, 0)),
            pl.BlockSpec((1, D), lambda i: (0, 0)),
        ],
        out_specs=pl.BlockSpec((1000, D), lambda i: (i, 0)),
        out_shape=jax.ShapeDtypeStruct((N, D), jnp.float32),
    )(agg, s0, s1, gamma2, beta2)


def kernel(x, edge_idx, edge_attr, W, b, gamma, beta):
    src = edge_idx[0]
    dst = edge_idx[1]
    pad = E_PAD - E
    ar = jnp.arange(pad, dtype=jnp.int32)
    src_p = jnp.concatenate([src, ar % N]).reshape(NW, NCH, C)
    dst_p = jnp.concatenate([dst, N + ar % (N_PAD - N)]).reshape(NW, NCH, C)
    attr_p = jnp.concatenate(
        [edge_attr, jnp.zeros((pad,), jnp.float32)]).reshape(NW, NCH, C)
    zdeg = jnp.zeros((RPT,), jnp.float32)
    zrows = jnp.zeros((RPT, D), jnp.float32)
    x_pad = jnp.pad(x, ((0, N_PAD - N), (0, 0)))

    degp = _deg_kernel(dst_p, attr_p, zdeg).reshape(2, N_PAD, 1)
    g, dis = _lin_call(x_pad, W, degp)
    sp = _scat_kernel(g, src_p, dst_p, attr_p, zrows).reshape(2, N_PAD, D)
    agg, s0, s1 = _agg_call(g, sp, dis, b.reshape(1, D))
    return _bn_call(agg, s0, s1, gamma.reshape(1, D), beta.reshape(1, D))


# speedup vs baseline: 1.3914x; 1.0245x over previous
import functools

import jax
import jax.numpy as jnp
from jax import lax
from jax.experimental import pallas as pl
from jax.experimental.pallas import tpu as pltpu
from jax.experimental.pallas import tpu_sc as plsc

N = 10000
N_PAD = 10240
E = 320000
D = 128
ALPHA = 0.2
EPS = 1e-5

NW = 32
C = 112
NCH = 90
E_PAD = NW * NCH * C
RPT = N_PAD // 16

_sc_mesh = plsc.VectorSubcoreMesh(core_axis_name="c", subcore_axis_name="s")


@functools.partial(
    pl.kernel,
    out_type=jax.ShapeDtypeStruct((2, 16, RPT), jnp.float32),
    mesh=_sc_mesh,
    scratch_types=[
        pltpu.VMEM((NCH, C), jnp.int32),
        pltpu.VMEM((NCH, C), jnp.float32),
        pltpu.VMEM_SHARED((N_PAD,), jnp.float32),
    ],
)
def _deg_kernel(dst_hbm, attr_hbm, zdeg_hbm, degp_hbm, dstbuf, attrbuf, deg_sh):
    c = lax.axis_index("c")
    s = lax.axis_index("s")
    wid = s * 2 + c
    pltpu.sync_copy(zdeg_hbm, deg_sh.at[pl.ds(s * RPT, RPT)])
    plsc.subcore_barrier()
    pltpu.sync_copy(dst_hbm.at[wid], dstbuf)
    pltpu.sync_copy(attr_hbm.at[wid], attrbuf)

    def chunk(j, carry):
        pltpu.sync_copy(attrbuf.at[j], deg_sh.at[dstbuf.at[j]], add=True)
        return carry

    lax.fori_loop(0, NCH, chunk, 0)
    plsc.subcore_barrier()
    pltpu.sync_copy(deg_sh.at[pl.ds(s * RPT, RPT)], degp_hbm.at[c, s])


def _lin_body(x_ref, w_ref, degp_ref, g_ref, dis_ref):
    h = jnp.dot(x_ref[...], w_ref[...], preferred_element_type=jnp.float32)
    deg = degp_ref[0] + degp_ref[1] + 1.0
    dis = lax.rsqrt(deg)
    g = h * dis
    g_ref[...] = g
    dis_ref[...] = dis


def _lin_call(x_pad, W, degp):
    return pl.pallas_call(
        _lin_body,
        grid=(16,),
        in_specs=[
            pl.BlockSpec((640, D), lambda i: (i, 0)),
            pl.BlockSpec((D, D), lambda i: (0, 0)),
            pl.BlockSpec((2, 640, 1), lambda i: (0, i, 0)),
        ],
        out_specs=[
            pl.BlockSpec((640, D), lambda i: (i, 0)),
            pl.BlockSpec((640, 1), lambda i: (i, 0)),
        ],
        out_shape=[
            jax.ShapeDtypeStruct((N_PAD, D), jnp.float32),
            jax.ShapeDtypeStruct((N_PAD, 1), jnp.float32),
        ],
    )(x_pad, W, degp)


@functools.partial(
    pl.kernel,
    out_type=jax.ShapeDtypeStruct((2, 16, RPT, D), jnp.float32),
    mesh=_sc_mesh,
    scratch_types=[
        pltpu.VMEM((2, C), jnp.int32),
        pltpu.VMEM((2, C), jnp.int32),
        pltpu.VMEM((2, C), jnp.float32),
        pltpu.VMEM((C, D), jnp.float32),
        pltpu.VMEM((C, D), jnp.float32),
        pltpu.VMEM((C, D), jnp.float32),
        pltpu.VMEM_SHARED((N_PAD, D), jnp.float32),
        pltpu.SemaphoreType.DMA,
        pltpu.SemaphoreType.DMA,
        pltpu.SemaphoreType.DMA,
        pltpu.SemaphoreType.DMA,
        pltpu.SemaphoreType.DMA,
        pltpu.SemaphoreType.DMA,
        pltpu.SemaphoreType.DMA,
        pltpu.SemaphoreType.DMA,
        pltpu.SemaphoreType.DMA,
    ],
)
def _scat_kernel(gp_hbm, src_hbm, dst_hbm, attr_hbm, zrows_hbm, sp_hbm,
                 srcbuf, dstbuf, attrbuf, rowsi0, rowsi1, rowsf, s_sh,
                 sem_src0, sem_src1, sem_dst0, sem_dst1, sem_attr0,
                 sem_attr1, sem_g0, sem_g1, sem_s):
    c = lax.axis_index("c")
    s = lax.axis_index("s")
    wid = s * 2 + c
    pltpu.sync_copy(zrows_hbm, s_sh.at[pl.ds(s * RPT, RPT)])
    plsc.subcore_barrier()

    rowsi = (rowsi0, rowsi1)
    sem_g = (sem_g0, sem_g1)
    sem_src = (sem_src0, sem_src1)
    sem_dst = (sem_dst0, sem_dst1)
    sem_attr = (sem_attr0, sem_attr1)

    def scale(p, bufi):
        def scale_grp(grp, carry2):
            base = grp * 16
            avs = attrbuf[p, pl.ds(base, 16)]
            for dr in range(16):
                av = jnp.full((16,), avs[dr], jnp.float32)
                for f in range(8):
                    sl = pl.ds(f * 16, 16)
                    rowsf[base + dr, sl] = bufi[base + dr, sl] * av
            return carry2

        lax.fori_loop(0, C // 16, scale_grp, 0)

    pltpu.async_copy(src_hbm.at[wid, 0], srcbuf.at[0], sem_src[0])
    pltpu.async_copy(src_hbm.at[wid, 1], srcbuf.at[1], sem_src[1])
    pltpu.async_copy(attr_hbm.at[wid, 0], attrbuf.at[0], sem_attr[0])
    pltpu.async_copy(attr_hbm.at[wid, 1], attrbuf.at[1], sem_attr[1])
    pltpu.async_copy(dst_hbm.at[wid, 0], dstbuf.at[0], sem_dst[0])
    pltpu.make_async_copy(
        src_hbm.at[wid, 0], srcbuf.at[0], sem_src[0]).wait()
    pltpu.async_copy(gp_hbm.at[srcbuf.at[0]], rowsi[0], sem_g[0])
    pltpu.make_async_copy(
        src_hbm.at[wid, 1], srcbuf.at[1], sem_src[1]).wait()
    pltpu.async_copy(gp_hbm.at[srcbuf.at[1]], rowsi[1], sem_g[1])

    def two(q, carry):
        for u in (0, 1):
            b = u
            nb = 1 - u
            j = q * 2 + u
            pltpu.make_async_copy(
                gp_hbm.at[srcbuf.at[b]], rowsi[b], sem_g[b]).wait()

            @pl.when(j < NCH - 2)
            def _():
                pltpu.async_copy(
                    src_hbm.at[wid, j + 2], srcbuf.at[b], sem_src[b])

            if u == 0:
                @pl.when(q > 0)
                def _():
                    pltpu.make_async_copy(
                        rowsf, s_sh.at[dstbuf.at[1]], sem_s).wait()
            else:
                pltpu.make_async_copy(
                    rowsf, s_sh.at[dstbuf.at[0]], sem_s).wait()

            @pl.when(j < NCH - 1)
            def _():
                pltpu.async_copy(
                    dst_hbm.at[wid, j + 1], dstbuf.at[nb], sem_dst[nb])

            pltpu.make_async_copy(
                attr_hbm.at[wid, j], attrbuf.at[b], sem_attr[b]).wait()
            scale(b, rowsi[b])
            pltpu.make_async_copy(
                dst_hbm.at[wid, j], dstbuf.at[b], sem_dst[b]).wait()
            pltpu.async_copy(rowsf, s_sh.at[dstbuf.at[b]], sem_s,
                             add=True)

            @pl.when(j < NCH - 2)
            def _():
                pltpu.make_async_copy(
                    src_hbm.at[wid, j + 2], srcbuf.at[b], sem_src[b]).wait()
                pltpu.async_copy(gp_hbm.at[srcbuf.at[b]], rowsi[b],
                                 sem_g[b])
                pltpu.async_copy(
                    attr_hbm.at[wid, j + 2], attrbuf.at[b], sem_attr[b])

        return carry

    lax.fori_loop(0, NCH // 2, two, 0)
    pltpu.make_async_copy(
        rowsf, s_sh.at[dstbuf.at[(NCH - 1) % 2]], sem_s).wait()
    plsc.subcore_barrier()
    pltpu.sync_copy(s_sh.at[pl.ds(s * RPT, RPT)], sp_hbm.at[c, s])


def _k4_body(g_ref, sp_ref, dis_ref, b_ref, gamma_ref, beta_ref, o_ref,
             agg_scr, sum_scr, sumsq_scr):
    i = pl.program_id(0)

    @pl.when(i < 10)
    def _():
        sblk = sp_ref[0] + sp_ref[1]
        agg = dis_ref[...] * (g_ref[...] + sblk) + b_ref[...]
        agg_scr[pl.ds(i * 1000, 1000), :] = agg
        s0 = jnp.sum(agg, axis=0, keepdims=True)
        s1 = jnp.sum(agg * agg, axis=0, keepdims=True)

        @pl.when(i == 0)
        def _():
            sum_scr[...] = s0
            sumsq_scr[...] = s1

        @pl.when(i > 0)
        def _():
            sum_scr[...] += s0
            sumsq_scr[...] += s1

    @pl.when(i >= 10)
    def _():
        mean = sum_scr[...] * (1.0 / N)
        var = sumsq_scr[...] * (1.0 / N) - mean * mean
        inv = lax.rsqrt(var + EPS)
        a = agg_scr[pl.ds((i - 10) * 1000, 1000), :]
        hn = (a - mean) * inv * gamma_ref[...] + beta_ref[...]
        o_ref[...] = jnp.where(hn >= 0, hn, ALPHA * hn)


def _k4_call(g, sp, dis, b2, gamma2, beta2):
    return pl.pallas_call(
        _k4_body,
        grid=(20,),
        in_specs=[
            pl.BlockSpec((1000, D), lambda i: (jnp.minimum(i, 9), 0)),
            pl.BlockSpec((2, 1000, D), lambda i: (0, jnp.minimum(i, 9), 0)),
            pl.BlockSpec((1000, 1), lambda i: (jnp.minimum(i, 9), 0)),
            pl.BlockSpec((1, D), lambda i: (0, 0)),
            pl.BlockSpec((1, D), lambda i: (0, 0)),
            pl.BlockSpec((1, D), lambda i: (0, 0)),
        ],
        out_specs=pl.BlockSpec((1000, D), lambda i: (jnp.maximum(i - 10, 0), 0)),
        out_shape=jax.ShapeDtypeStruct((N, D), jnp.float32),
        scratch_shapes=[
            pltpu.VMEM((N, D), jnp.float32),
            pltpu.VMEM((1, D), jnp.float32),
            pltpu.VMEM((1, D), jnp.float32),
        ],
    )(g, sp, dis, b2, gamma2, beta2)


def kernel(x, edge_idx, edge_attr, W, b, gamma, beta):
    src = edge_idx[0]
    dst = edge_idx[1]
    pad = E_PAD - E
    ar = jnp.arange(pad, dtype=jnp.int32)
    src_p = jnp.concatenate([src, ar % N]).reshape(NW, NCH, C)
    dst_p = jnp.concatenate([dst, N + ar % (N_PAD - N)]).reshape(NW, NCH, C)
    attr_p = jnp.concatenate(
        [edge_attr, jnp.zeros((pad,), jnp.float32)]).reshape(NW, NCH, C)
    zdeg = jnp.zeros((RPT,), jnp.float32)
    zrows = jnp.zeros((RPT, D), jnp.float32)
    x_pad = jnp.pad(x, ((0, N_PAD - N), (0, 0)))

    degp = _deg_kernel(dst_p, attr_p, zdeg).reshape(2, N_PAD, 1)
    g, dis = _lin_call(x_pad, W, degp)
    sp = _scat_kernel(g, src_p, dst_p, attr_p, zrows).reshape(2, N_PAD, D)
    return _k4_call(g, sp, dis, b.reshape(1, D),
                    gamma.reshape(1, D), beta.reshape(1, D))


# speedup vs baseline: 1.4249x; 1.0240x over previous
import functools

import jax
import jax.numpy as jnp
from jax import lax
from jax.experimental import pallas as pl
from jax.experimental.pallas import tpu as pltpu
from jax.experimental.pallas import tpu_sc as plsc

N = 10000
N_PAD = 10240
E = 320000
D = 128
ALPHA = 0.2
EPS = 1e-5

NW = 32
C = 112
NCH = 90
E_PAD = NW * NCH * C
RPT = N_PAD // 16

_sc_mesh = plsc.VectorSubcoreMesh(core_axis_name="c", subcore_axis_name="s")


@functools.partial(
    pl.kernel,
    out_type=jax.ShapeDtypeStruct((2, 16, RPT), jnp.float32),
    mesh=_sc_mesh,
    scratch_types=[
        pltpu.VMEM((NCH, C), jnp.int32),
        pltpu.VMEM((NCH, C), jnp.float32),
        pltpu.VMEM_SHARED((N_PAD,), jnp.float32),
        pltpu.SemaphoreType.DMA,
    ],
)
def _deg_kernel(dst_hbm, attr_hbm, zdeg_hbm, degp_hbm, dstbuf, attrbuf,
                deg_sh, sem_d):
    c = lax.axis_index("c")
    s = lax.axis_index("s")
    wid = s * 2 + c
    pltpu.sync_copy(zdeg_hbm, deg_sh.at[pl.ds(s * RPT, RPT)])
    plsc.subcore_barrier()
    pltpu.sync_copy(dst_hbm.at[wid], dstbuf)
    pltpu.sync_copy(attr_hbm.at[wid], attrbuf)

    def chunk(j, carry):
        pltpu.async_copy(attrbuf.at[j], deg_sh.at[dstbuf.at[j]], sem_d,
                         add=True)

        @pl.when(j >= 8)
        def _():
            pltpu.make_async_copy(
                attrbuf.at[j - 8], deg_sh.at[dstbuf.at[j - 8]], sem_d).wait()

        return carry

    lax.fori_loop(0, NCH, chunk, 0)

    def drain(j, carry):
        pltpu.make_async_copy(
            attrbuf.at[j], deg_sh.at[dstbuf.at[j]], sem_d).wait()
        return carry

    lax.fori_loop(NCH - 8, NCH, drain, 0)
    plsc.subcore_barrier()
    pltpu.sync_copy(deg_sh.at[pl.ds(s * RPT, RPT)], degp_hbm.at[c, s])


def _lin_body(x_ref, w_ref, degp_ref, g_ref, dis_ref):
    h = jnp.dot(x_ref[...], w_ref[...], preferred_element_type=jnp.float32)
    deg = degp_ref[0] + degp_ref[1] + 1.0
    dis = lax.rsqrt(deg)
    g = h * dis
    g_ref[...] = g
    dis_ref[...] = dis


def _lin_call(x_pad, W, degp):
    return pl.pallas_call(
        _lin_body,
        grid=(16,),
        in_specs=[
            pl.BlockSpec((640, D), lambda i: (i, 0)),
            pl.BlockSpec((D, D), lambda i: (0, 0)),
            pl.BlockSpec((2, 640, 1), lambda i: (0, i, 0)),
        ],
        out_specs=[
            pl.BlockSpec((640, D), lambda i: (i, 0)),
            pl.BlockSpec((640, 1), lambda i: (i, 0)),
        ],
        out_shape=[
            jax.ShapeDtypeStruct((N_PAD, D), jnp.float32),
            jax.ShapeDtypeStruct((N_PAD, 1), jnp.float32),
        ],
    )(x_pad, W, degp)


@functools.partial(
    pl.kernel,
    out_type=jax.ShapeDtypeStruct((2, 16, RPT, D), jnp.float32),
    mesh=_sc_mesh,
    scratch_types=[
        pltpu.VMEM((2, C), jnp.int32),
        pltpu.VMEM((2, C), jnp.int32),
        pltpu.VMEM((2, C), jnp.float32),
        pltpu.VMEM((C, D), jnp.float32),
        pltpu.VMEM((C, D), jnp.float32),
        pltpu.VMEM((C, D), jnp.float32),
        pltpu.VMEM_SHARED((N_PAD, D), jnp.float32),
        pltpu.SemaphoreType.DMA,
        pltpu.SemaphoreType.DMA,
        pltpu.SemaphoreType.DMA,
        pltpu.SemaphoreType.DMA,
        pltpu.SemaphoreType.DMA,
        pltpu.SemaphoreType.DMA,
        pltpu.SemaphoreType.DMA,
        pltpu.SemaphoreType.DMA,
        pltpu.SemaphoreType.DMA,
    ],
)
def _scat_kernel(gp_hbm, src_hbm, dst_hbm, attr_hbm, zrows_hbm, sp_hbm,
                 srcbuf, dstbuf, attrbuf, rowsi0, rowsi1, rowsf, s_sh,
                 sem_src0, sem_src1, sem_dst0, sem_dst1, sem_attr0,
                 sem_attr1, sem_g0, sem_g1, sem_s):
    c = lax.axis_index("c")
    s = lax.axis_index("s")
    wid = s * 2 + c
    pltpu.sync_copy(zrows_hbm, s_sh.at[pl.ds(s * RPT, RPT)])
    plsc.subcore_barrier()

    rowsi = (rowsi0, rowsi1)
    sem_g = (sem_g0, sem_g1)
    sem_src = (sem_src0, sem_src1)
    sem_dst = (sem_dst0, sem_dst1)
    sem_attr = (sem_attr0, sem_attr1)

    def scale(p, bufi):
        def scale_grp(grp, carry2):
            base = grp * 16
            avs = attrbuf[p, pl.ds(base, 16)]
            for dr in range(16):
                av = jnp.full((16,), avs[dr], jnp.float32)
                for f in range(8):
                    sl = pl.ds(f * 16, 16)
                    rowsf[base + dr, sl] = bufi[base + dr, sl] * av
            return carry2

        lax.fori_loop(0, C // 16, scale_grp, 0)

    pltpu.async_copy(src_hbm.at[wid, 0], srcbuf.at[0], sem_src[0])
    pltpu.async_copy(src_hbm.at[wid, 1], srcbuf.at[1], sem_src[1])
    pltpu.async_copy(attr_hbm.at[wid, 0], attrbuf.at[0], sem_attr[0])
    pltpu.async_copy(attr_hbm.at[wid, 1], attrbuf.at[1], sem_attr[1])
    pltpu.async_copy(dst_hbm.at[wid, 0], dstbuf.at[0], sem_dst[0])
    pltpu.make_async_copy(
        src_hbm.at[wid, 0], srcbuf.at[0], sem_src[0]).wait()
    pltpu.async_copy(gp_hbm.at[srcbuf.at[0]], rowsi[0], sem_g[0])
    pltpu.make_async_copy(
        src_hbm.at[wid, 1], srcbuf.at[1], sem_src[1]).wait()
    pltpu.async_copy(gp_hbm.at[srcbuf.at[1]], rowsi[1], sem_g[1])

    def two(q, carry):
        for u in (0, 1):
            b = u
            nb = 1 - u
            j = q * 2 + u
            pltpu.make_async_copy(
                gp_hbm.at[srcbuf.at[b]], rowsi[b], sem_g[b]).wait()

            @pl.when(j < NCH - 2)
            def _():
                pltpu.async_copy(
                    src_hbm.at[wid, j + 2], srcbuf.at[b], sem_src[b])

            if u == 0:
                @pl.when(q > 0)
                def _():
                    pltpu.make_async_copy(
                        rowsf, s_sh.at[dstbuf.at[1]], sem_s).wait()
            else:
                pltpu.make_async_copy(
                    rowsf, s_sh.at[dstbuf.at[0]], sem_s).wait()

            @pl.when(j < NCH - 1)
            def _():
                pltpu.async_copy(
                    dst_hbm.at[wid, j + 1], dstbuf.at[nb], sem_dst[nb])

            pltpu.make_async_copy(
                attr_hbm.at[wid, j], attrbuf.at[b], sem_attr[b]).wait()
            scale(b, rowsi[b])
            pltpu.make_async_copy(
                dst_hbm.at[wid, j], dstbuf.at[b], sem_dst[b]).wait()
            pltpu.async_copy(rowsf, s_sh.at[dstbuf.at[b]], sem_s,
                             add=True)

            @pl.when(j < NCH - 2)
            def _():
                pltpu.make_async_copy(
                    src_hbm.at[wid, j + 2], srcbuf.at[b], sem_src[b]).wait()
                pltpu.async_copy(gp_hbm.at[srcbuf.at[b]], rowsi[b],
                                 sem_g[b])
                pltpu.async_copy(
                    attr_hbm.at[wid, j + 2], attrbuf.at[b], sem_attr[b])

        return carry

    lax.fori_loop(0, NCH // 2, two, 0)
    pltpu.make_async_copy(
        rowsf, s_sh.at[dstbuf.at[(NCH - 1) % 2]], sem_s).wait()
    plsc.subcore_barrier()
    pltpu.sync_copy(s_sh.at[pl.ds(s * RPT, RPT)], sp_hbm.at[c, s])


def _k4_body(g_ref, sp_ref, dis_ref, b_ref, gamma_ref, beta_ref, o_ref,
             agg_scr, sum_scr, sumsq_scr):
    i = pl.program_id(0)

    @pl.when(i < 10)
    def _():
        sblk = sp_ref[0] + sp_ref[1]
        agg = dis_ref[...] * (g_ref[...] + sblk) + b_ref[...]
        agg_scr[pl.ds(i * 1000, 1000), :] = agg
        s0 = jnp.sum(agg, axis=0, keepdims=True)
        s1 = jnp.sum(agg * agg, axis=0, keepdims=True)

        @pl.when(i == 0)
        def _():
            sum_scr[...] = s0
            sumsq_scr[...] = s1

        @pl.when(i > 0)
        def _():
            sum_scr[...] += s0
            sumsq_scr[...] += s1

    @pl.when(i >= 10)
    def _():
        mean = sum_scr[...] * (1.0 / N)
        var = sumsq_scr[...] * (1.0 / N) - mean * mean
        inv = lax.rsqrt(var + EPS)
        a = agg_scr[pl.ds((i - 10) * 1000, 1000), :]
        hn = (a - mean) * inv * gamma_ref[...] + beta_ref[...]
        o_ref[...] = jnp.where(hn >= 0, hn, ALPHA * hn)


def _k4_call(g, sp, dis, b2, gamma2, beta2):
    return pl.pallas_call(
        _k4_body,
        grid=(20,),
        in_specs=[
            pl.BlockSpec((1000, D), lambda i: (jnp.minimum(i, 9), 0)),
            pl.BlockSpec((2, 1000, D), lambda i: (0, jnp.minimum(i, 9), 0)),
            pl.BlockSpec((1000, 1), lambda i: (jnp.minimum(i, 9), 0)),
            pl.BlockSpec((1, D), lambda i: (0, 0)),
            pl.BlockSpec((1, D), lambda i: (0, 0)),
            pl.BlockSpec((1, D), lambda i: (0, 0)),
        ],
        out_specs=pl.BlockSpec((1000, D), lambda i: (jnp.maximum(i - 10, 0), 0)),
        out_shape=jax.ShapeDtypeStruct((N, D), jnp.float32),
        scratch_shapes=[
            pltpu.VMEM((N, D), jnp.float32),
            pltpu.VMEM((1, D), jnp.float32),
            pltpu.VMEM((1, D), jnp.float32),
        ],
    )(g, sp, dis, b2, gamma2, beta2)


def kernel(x, edge_idx, edge_attr, W, b, gamma, beta):
    src = edge_idx[0]
    dst = edge_idx[1]
    pad = E_PAD - E
    ar = jnp.arange(pad, dtype=jnp.int32)
    src_p = jnp.concatenate([src, ar % N]).reshape(NW, NCH, C)
    dst_p = jnp.concatenate([dst, N + ar % (N_PAD - N)]).reshape(NW, NCH, C)
    attr_p = jnp.concatenate(
        [edge_attr, jnp.zeros((pad,), jnp.float32)]).reshape(NW, NCH, C)
    zdeg = jnp.zeros((RPT,), jnp.float32)
    zrows = jnp.zeros((RPT, D), jnp.float32)
    x_pad = jnp.pad(x, ((0, N_PAD - N), (0, 0)))

    degp = _deg_kernel(dst_p, attr_p, zdeg).reshape(2, N_PAD, 1)
    g, dis = _lin_call(x_pad, W, degp)
    sp = _scat_kernel(g, src_p, dst_p, attr_p, zrows).reshape(2, N_PAD, D)
    return _k4_call(g, sp, dis, b.reshape(1, D),
                    gamma.reshape(1, D), beta.reshape(1, D))
